# Initial kernel scaffold; baseline (speedup 1.0000x reference)
import jax
import jax.numpy as jnp
from jax.experimental import pallas as pl


def kernel(x, edge_index, W1, b1, W2, b2, W3, b3):
    raise NotImplementedError("write your pallas kernel here")


# speedup vs baseline: 13.3911x; 13.3911x over previous
import functools

import jax
import jax.numpy as jnp
from jax import lax
from jax.experimental import pallas as pl
from jax.experimental.pallas import tpu as pltpu
from jax.experimental.pallas import tpu_sc as plsc

NN = 10000
NP = 10240
DD = 128
EE = 320000
NC = 2
NS = 16
EPC = EE // NC
CH = 128
CPC = EPC // CH
RPT = NP // NS

_mesh = plsc.VectorSubcoreMesh(core_axis_name="c", subcore_axis_name="s")

_FULL, _EXTRA = CPC // NS, CPC % NS


def _num_chunks(s):
    return _FULL + jnp.where(s < _EXTRA, 1, 0)


def _deg_hist(dst):
    @functools.partial(
        pl.kernel,
        out_type=jax.ShapeDtypeStruct((NC, NP), jnp.float32),
        mesh=_mesh,
        scratch_types=[
            pltpu.VMEM_SHARED((NP,), jnp.float32),
            pltpu.VMEM((1, CH), jnp.int32),
            pltpu.VMEM((CH,), jnp.float32),
            pltpu.VMEM((RPT,), jnp.float32),
        ],
    )
    def k(dst_hbm, out_hbm, hist_sh, idx_v, ones_v, zb_v):
        c = lax.axis_index("c")
        s = lax.axis_index("s")
        for i in range(CH // 16):
            ones_v[pl.ds(i * 16, 16)] = jnp.ones((16,), jnp.float32)

        @pl.loop(0, RPT // 16)
        def _(i):
            zb_v[pl.ds(i * 16, 16)] = jnp.zeros((16,), jnp.float32)

        pltpu.sync_copy(zb_v, hist_sh.at[pl.ds(s * RPT, RPT)])
        plsc.subcore_barrier()

        @pl.loop(0, _num_chunks(s))
        def _(j):
            off = c * EPC + (j * NS + s) * CH
            pltpu.sync_copy(dst_hbm.at[pl.ds(off, CH)], idx_v.at[0])
            pltpu.sync_copy(ones_v, hist_sh.at[idx_v.at[0]], add=True)

        plsc.subcore_barrier()
        pltpu.sync_copy(hist_sh.at[pl.ds(s * RPT, RPT)],
                        out_hbm.at[c, pl.ds(s * RPT, RPT)])

    return k(dst)


def _edge_agg(hs, src, dst):
    @functools.partial(
        pl.kernel,
        out_type=jax.ShapeDtypeStruct((NC, NP, DD), jnp.float32),
        mesh=_mesh,
        scratch_types=[
            pltpu.VMEM_SHARED((NP, DD), jnp.float32),
            pltpu.VMEM((1, CH), jnp.int32),
            pltpu.VMEM((1, CH), jnp.int32),
            pltpu.VMEM((CH, DD), jnp.float32),
            pltpu.SemaphoreType.DMA,
        ],
    )
    def k(hs_hbm, src_hbm, dst_hbm, out_hbm, acc_sh, si_v, di_v, rows_v, sem):
        c = lax.axis_index("c")
        s = lax.axis_index("s")

        @pl.loop(0, CH)
        def _(r):
            for i in range(DD // 16):
                rows_v[r, pl.ds(i * 16, 16)] = jnp.zeros((16,), jnp.float32)

        @pl.loop(0, RPT // CH)
        def _(t):
            pltpu.sync_copy(rows_v, acc_sh.at[pl.ds(s * RPT + t * CH, CH)])

        plsc.subcore_barrier()

        @pl.loop(0, _num_chunks(s))
        def _(j):
            off = c * EPC + (j * NS + s) * CH
            pltpu.sync_copy(src_hbm.at[pl.ds(off, CH)], si_v.at[0])
            pltpu.sync_copy(dst_hbm.at[pl.ds(off, CH)], di_v.at[0])
            pltpu.async_copy(hs_hbm.at[si_v.at[0]], rows_v, sem).wait()
            pltpu.sync_copy(rows_v, acc_sh.at[di_v.at[0]], add=True)

        plsc.subcore_barrier()
        pltpu.sync_copy(acc_sh.at[pl.ds(s * RPT, RPT)],
                        out_hbm.at[c, pl.ds(s * RPT, RPT)])

    return k(hs, src, dst)


_BN = 512


def _mm_body(x_ref, w_ref, o_ref):
    o_ref[...] = jnp.dot(x_ref[...], w_ref[...],
                         preferred_element_type=jnp.float32)


def _matmul(xp, w):
    return pl.pallas_call(
        _mm_body,
        grid=(NP // _BN,),
        in_specs=[pl.BlockSpec((_BN, DD), lambda i: (i, 0)),
                  pl.BlockSpec((DD, DD), lambda i: (0, 0))],
        out_specs=pl.BlockSpec((_BN, DD), lambda i: (i, 0)),
        out_shape=jax.ShapeDtypeStruct((NP, DD), jnp.float32),
    )(xp, w)


def _deg_body(h0_ref, h1_ref, hh_ref, disb_ref, hs_ref):
    deg = h0_ref[...] + h1_ref[...] + 1.0
    db = jnp.broadcast_to(lax.rsqrt(deg), (_BN, DD))
    disb_ref[...] = db
    hs_ref[...] = db * hh_ref[...]


def _finish_deg(hist0, hist1, h1):
    return pl.pallas_call(
        _deg_body,
        grid=(NP // _BN,),
        in_specs=[pl.BlockSpec((_BN, 1), lambda i: (i, 0)),
                  pl.BlockSpec((_BN, 1), lambda i: (i, 0)),
                  pl.BlockSpec((_BN, DD), lambda i: (i, 0))],
        out_specs=[pl.BlockSpec((_BN, DD), lambda i: (i, 0)),
                   pl.BlockSpec((_BN, DD), lambda i: (i, 0))],
        out_shape=[jax.ShapeDtypeStruct((NP, DD), jnp.float32),
                   jax.ShapeDtypeStruct((NP, DD), jnp.float32)],
    )(hist0, hist1, h1)


def _mid_body(p0_ref, p1_ref, hs_ref, db_ref, b_ref, w_ref, o_ref):
    t = db_ref[...] * (p0_ref[...] + p1_ref[...] + hs_ref[...]) + b_ref[...]
    a = jnp.tanh(t)
    o_ref[...] = db_ref[...] * jnp.dot(a, w_ref[...],
                                       preferred_element_type=jnp.float32)


def _mid(p0, p1, hs, disb, b, w):
    return pl.pallas_call(
        _mid_body,
        grid=(NP // _BN,),
        in_specs=[pl.BlockSpec((_BN, DD), lambda i: (i, 0)),
                  pl.BlockSpec((_BN, DD), lambda i: (i, 0)),
                  pl.BlockSpec((_BN, DD), lambda i: (i, 0)),
                  pl.BlockSpec((_BN, DD), lambda i: (i, 0)),
                  pl.BlockSpec((1, DD), lambda i: (0, 0)),
                  pl.BlockSpec((DD, DD), lambda i: (0, 0))],
        out_specs=pl.BlockSpec((_BN, DD), lambda i: (i, 0)),
        out_shape=jax.ShapeDtypeStruct((NP, DD), jnp.float32),
    )(p0, p1, hs, disb, b, w)


def _fin_body(p0_ref, p1_ref, hs_ref, db_ref, b_ref, o_ref):
    t = db_ref[...] * (p0_ref[...] + p1_ref[...] + hs_ref[...]) + b_ref[...]
    m = jnp.max(t, axis=1, keepdims=True)
    e = jnp.exp(t - m)
    ssum = jnp.sum(e, axis=1, keepdims=True)
    o_ref[...] = t - m - jnp.log(ssum)


def _fin(p0, p1, hs, disb, b):
    return pl.pallas_call(
        _fin_body,
        grid=(NP // _BN,),
        in_specs=[pl.BlockSpec((_BN, DD), lambda i: (i, 0)),
                  pl.BlockSpec((_BN, DD), lambda i: (i, 0)),
                  pl.BlockSpec((_BN, DD), lambda i: (i, 0)),
                  pl.BlockSpec((_BN, DD), lambda i: (i, 0)),
                  pl.BlockSpec((1, DD), lambda i: (0, 0))],
        out_specs=pl.BlockSpec((_BN, DD), lambda i: (i, 0)),
        out_shape=jax.ShapeDtypeStruct((NP, DD), jnp.float32),
    )(p0, p1, hs, disb, b)


def kernel(x, edge_index, W1, b1, W2, b2, W3, b3):
    assert x.shape == (NN, DD) and edge_index.shape == (2, EE)
    src = edge_index[0]
    dst = edge_index[1]
    xp = jnp.pad(x, ((0, NP - NN), (0, 0)))

    hist = _deg_hist(dst)
    h1 = _matmul(xp, W1)
    disb, hs1 = _finish_deg(hist[0].reshape(NP, 1),
                            hist[1].reshape(NP, 1), h1)

    p = _edge_agg(hs1, src, dst)
    hs2 = _mid(p[0], p[1], hs1, disb, b1.reshape(1, DD), W2)
    p = _edge_agg(hs2, src, dst)
    hs3 = _mid(p[0], p[1], hs2, disb, b2.reshape(1, DD), W3)
    p = _edge_agg(hs3, src, dst)
    out = _fin(p[0], p[1], hs3, disb, b3.reshape(1, DD))
    return out[:NN]


# speedup vs baseline: 24.1886x; 1.8063x over previous
import functools

import jax
import jax.numpy as jnp
from jax import lax
from jax.experimental import pallas as pl
from jax.experimental.pallas import tpu as pltpu
from jax.experimental.pallas import tpu_sc as plsc

NN = 10000
NP = 10240
DD = 128
EE = 320000
NC = 2
NS = 16
NW = NC * NS
CH = 128
RPT_E = 80
RR = RPT_E * NW
EPAD = RR * CH
RPT = NP // NS

_mesh = plsc.VectorSubcoreMesh(core_axis_name="c", subcore_axis_name="s")


def _deg_hist(dstp):
    @functools.partial(
        pl.kernel,
        out_type=jax.ShapeDtypeStruct((NC, NP), jnp.float32),
        mesh=_mesh,
        scratch_types=[
            pltpu.VMEM_SHARED((NP,), jnp.float32),
            pltpu.VMEM((RPT_E, CH), jnp.int32),
            pltpu.VMEM((CH,), jnp.float32),
            pltpu.VMEM((RPT,), jnp.float32),
        ],
    )
    def k(dst_hbm, out_hbm, hist_sh, di_all, ones_v, zb_v):
        c = lax.axis_index("c")
        s = lax.axis_index("s")
        row0 = (c * NS + s) * RPT_E
        for i in range(CH // 16):
            ones_v[pl.ds(i * 16, 16)] = jnp.ones((16,), jnp.float32)

        @pl.loop(0, RPT // 16)
        def _(i):
            zb_v[pl.ds(i * 16, 16)] = jnp.zeros((16,), jnp.float32)

        pltpu.sync_copy(zb_v, hist_sh.at[pl.ds(s * RPT, RPT)])
        pltpu.sync_copy(dst_hbm.at[pl.ds(row0, RPT_E)], di_all)
        plsc.subcore_barrier()

        @pl.loop(0, RPT_E)
        def _(j):
            pltpu.sync_copy(ones_v, hist_sh.at[di_all.at[j]], add=True)

        plsc.subcore_barrier()
        pltpu.sync_copy(hist_sh.at[pl.ds(s * RPT, RPT)],
                        out_hbm.at[c, pl.ds(s * RPT, RPT)])

    return k(dstp)


def _edge_agg(hs, srcp, dstp):
    RB = 16

    @functools.partial(
        pl.kernel,
        out_type=jax.ShapeDtypeStruct((NC, NP, DD), jnp.float32),
        mesh=_mesh,
        scratch_types=[
            pltpu.VMEM_SHARED((NP, DD), jnp.float32),
            pltpu.VMEM((RB, CH), jnp.int32),
            pltpu.VMEM((RB, CH), jnp.int32),
            pltpu.VMEM((2, CH, DD), jnp.float32),
            pltpu.SemaphoreType.DMA,
            pltpu.SemaphoreType.DMA,
        ],
    )
    def k(hs_hbm, src_hbm, dst_hbm, out_hbm, acc_sh, si_blk, di_blk, rows_v,
          sem0, sem1):
        c = lax.axis_index("c")
        s = lax.axis_index("s")
        row0 = (c * NS + s) * RPT_E

        @pl.loop(0, CH)
        def _(r):
            for i in range(DD // 16):
                rows_v[0, r, pl.ds(i * 16, 16)] = jnp.zeros((16,), jnp.float32)

        @pl.loop(0, RPT // CH)
        def _(t):
            pltpu.sync_copy(rows_v.at[0], acc_sh.at[pl.ds(s * RPT + t * CH, CH)])

        plsc.subcore_barrier()

        def wait_gather(b, sem):
            pltpu.make_async_copy(hs_hbm.at[pl.ds(0, CH)], rows_v.at[b],
                                  sem).wait()

        @pl.loop(0, RPT_E // RB)
        def _(bk):
            pltpu.sync_copy(src_hbm.at[pl.ds(row0 + bk * RB, RB)], si_blk)
            pltpu.sync_copy(dst_hbm.at[pl.ds(row0 + bk * RB, RB)], di_blk)
            pltpu.async_copy(hs_hbm.at[si_blk.at[0]], rows_v.at[0], sem0)

            @pl.loop(0, RB - 2, step=2)
            def _(j):
                pltpu.async_copy(hs_hbm.at[si_blk.at[j + 1]], rows_v.at[1],
                                 sem1)
                wait_gather(0, sem0)
                pltpu.sync_copy(rows_v.at[0], acc_sh.at[di_blk.at[j]],
                                add=True)
                pltpu.async_copy(hs_hbm.at[si_blk.at[j + 2]], rows_v.at[0],
                                 sem0)
                wait_gather(1, sem1)
                pltpu.sync_copy(rows_v.at[1], acc_sh.at[di_blk.at[j + 1]],
                                add=True)

            pltpu.async_copy(hs_hbm.at[si_blk.at[RB - 1]], rows_v.at[1], sem1)
            wait_gather(0, sem0)
            pltpu.sync_copy(rows_v.at[0], acc_sh.at[di_blk.at[RB - 2]],
                            add=True)
            wait_gather(1, sem1)
            pltpu.sync_copy(rows_v.at[1], acc_sh.at[di_blk.at[RB - 1]],
                            add=True)

        plsc.subcore_barrier()
        pltpu.sync_copy(acc_sh.at[pl.ds(s * RPT, RPT)],
                        out_hbm.at[c, pl.ds(s * RPT, RPT)])

    return k(hs, srcp, dstp)


_BN = 512


def _mm_body(x_ref, w_ref, o_ref):
    o_ref[...] = jnp.dot(x_ref[...], w_ref[...],
                         preferred_element_type=jnp.float32)


def _matmul(xp, w):
    return pl.pallas_call(
        _mm_body,
        grid=(NP // _BN,),
        in_specs=[pl.BlockSpec((_BN, DD), lambda i: (i, 0)),
                  pl.BlockSpec((DD, DD), lambda i: (0, 0))],
        out_specs=pl.BlockSpec((_BN, DD), lambda i: (i, 0)),
        out_shape=jax.ShapeDtypeStruct((NP, DD), jnp.float32),
    )(xp, w)


def _deg_body(h0_ref, h1_ref, hh_ref, disb_ref, hs_ref):
    deg = h0_ref[...] + h1_ref[...] + 1.0
    db = jnp.broadcast_to(lax.rsqrt(deg), (_BN, DD))
    disb_ref[...] = db
    hs_ref[...] = db * hh_ref[...]


def _finish_deg(hist0, hist1, h1):
    return pl.pallas_call(
        _deg_body,
        grid=(NP // _BN,),
        in_specs=[pl.BlockSpec((_BN, 1), lambda i: (i, 0)),
                  pl.BlockSpec((_BN, 1), lambda i: (i, 0)),
                  pl.BlockSpec((_BN, DD), lambda i: (i, 0))],
        out_specs=[pl.BlockSpec((_BN, DD), lambda i: (i, 0)),
                   pl.BlockSpec((_BN, DD), lambda i: (i, 0))],
        out_shape=[jax.ShapeDtypeStruct((NP, DD), jnp.float32),
                   jax.ShapeDtypeStruct((NP, DD), jnp.float32)],
    )(hist0, hist1, h1)


def _mid_body(p0_ref, p1_ref, hs_ref, db_ref, b_ref, w_ref, o_ref):
    t = db_ref[...] * (p0_ref[...] + p1_ref[...] + hs_ref[...]) + b_ref[...]
    a = jnp.tanh(t)
    o_ref[...] = db_ref[...] * jnp.dot(a, w_ref[...],
                                       preferred_element_type=jnp.float32)


def _mid(p0, p1, hs, disb, b, w):
    return pl.pallas_call(
        _mid_body,
        grid=(NP // _BN,),
        in_specs=[pl.BlockSpec((_BN, DD), lambda i: (i, 0)),
                  pl.BlockSpec((_BN, DD), lambda i: (i, 0)),
                  pl.BlockSpec((_BN, DD), lambda i: (i, 0)),
                  pl.BlockSpec((_BN, DD), lambda i: (i, 0)),
                  pl.BlockSpec((1, DD), lambda i: (0, 0)),
                  pl.BlockSpec((DD, DD), lambda i: (0, 0))],
        out_specs=pl.BlockSpec((_BN, DD), lambda i: (i, 0)),
        out_shape=jax.ShapeDtypeStruct((NP, DD), jnp.float32),
    )(p0, p1, hs, disb, b, w)


def _fin_body(p0_ref, p1_ref, hs_ref, db_ref, b_ref, o_ref):
    t = db_ref[...] * (p0_ref[...] + p1_ref[...] + hs_ref[...]) + b_ref[...]
    m = jnp.max(t, axis=1, keepdims=True)
    e = jnp.exp(t - m)
    ssum = jnp.sum(e, axis=1, keepdims=True)
    o_ref[...] = t - m - jnp.log(ssum)


def _fin(p0, p1, hs, disb, b):
    return pl.pallas_call(
        _fin_body,
        grid=(NP // _BN,),
        in_specs=[pl.BlockSpec((_BN, DD), lambda i: (i, 0)),
                  pl.BlockSpec((_BN, DD), lambda i: (i, 0)),
                  pl.BlockSpec((_BN, DD), lambda i: (i, 0)),
                  pl.BlockSpec((_BN, DD), lambda i: (i, 0)),
                  pl.BlockSpec((1, DD), lambda i: (0, 0))],
        out_specs=pl.BlockSpec((_BN, DD), lambda i: (i, 0)),
        out_shape=jax.ShapeDtypeStruct((NP, DD), jnp.float32),
    )(p0, p1, hs, disb, b)


def kernel(x, edge_index, W1, b1, W2, b2, W3, b3):
    assert x.shape == (NN, DD) and edge_index.shape == (2, EE)
    pad_idx = (jnp.arange(EPAD - EE, dtype=jnp.int32) % (NP - NN)) + NN
    src = jnp.concatenate([edge_index[0], pad_idx]).reshape(RR, CH)
    dst = jnp.concatenate([edge_index[1], pad_idx]).reshape(RR, CH)
    xp = jnp.pad(x, ((0, NP - NN), (0, 0)))

    hist = _deg_hist(dst)
    h1 = _matmul(xp, W1)
    disb, hs1 = _finish_deg(hist[0].reshape(NP, 1),
                            hist[1].reshape(NP, 1), h1)

    p = _edge_agg(hs1, src, dst)
    hs2 = _mid(p[0], p[1], hs1, disb, b1.reshape(1, DD), W2)
    p = _edge_agg(hs2, src, dst)
    hs3 = _mid(p[0], p[1], hs2, disb, b2.reshape(1, DD), W3)
    p = _edge_agg(hs3, src, dst)
    out = _fin(p[0], p[1], hs3, disb, b3.reshape(1, DD))
    return out[:NN]
